# Initial kernel scaffold; baseline (speedup 1.0000x reference)
import jax
import jax.numpy as jnp
from jax.experimental import pallas as pl


def kernel(inp, w_gate, w_noise, noise):
    raise NotImplementedError("write your pallas kernel here")


# speedup vs baseline: 2.7999x; 2.7999x over previous
import functools

import jax
import jax.numpy as jnp
from jax import lax
from jax.experimental import pallas as pl
from jax.experimental.pallas import tpu as pltpu

D_MODEL = 1024
NUM_EXPERT = 8
TOP_K = 2
N_TOKENS = 32768
NOISE_EPS = 0.01

BLK = 2048
NEG = -jnp.inf


def _ncdf(z):
    x = z * 0.7071067811865476
    a = jnp.abs(x)
    t = 1.0 / (1.0 + 0.3275911 * a)
    poly = t * (0.254829592 + t * (-0.284496736 + t * (1.421413741
                + t * (-1.453152027 + t * 1.061405429))))
    erf_a = 1.0 - poly * jnp.exp(-a * a)
    return 0.5 * (1.0 + jnp.where(x < 0, -erf_a, erf_a))


def _router_kernel(inp_ref, w_ref, noise_ref, idx_ref, gates_ref, loss_ref,
                   acc_ref):
    i = pl.program_id(0)

    @pl.when(i == 0)
    def _():
        acc_ref[...] = jnp.zeros_like(acc_ref)

    logits16 = jnp.dot(inp_ref[...], w_ref[...],
                       preferred_element_type=jnp.float32)
    clean = logits16[:, :NUM_EXPERT]
    raw = logits16[:, NUM_EXPERT:]
    stddev = (jnp.maximum(raw, 0.0)
              + jnp.log(1.0 + jnp.exp(-jnp.abs(raw))) + NOISE_EPS)
    noisy = clean + noise_ref[...] * stddev

    eio = lax.broadcasted_iota(jnp.int32, (BLK, NUM_EXPERT), 1)
    m1 = jnp.max(noisy, axis=1, keepdims=True)
    idx1 = jnp.min(jnp.where(noisy == m1, eio, NUM_EXPERT), axis=1,
                   keepdims=True)
    v2 = jnp.where(eio == idx1, NEG, noisy)
    m2 = jnp.max(v2, axis=1, keepdims=True)
    idx2 = jnp.min(jnp.where(v2 == m2, eio, NUM_EXPERT), axis=1,
                   keepdims=True)
    v3 = jnp.where(eio == idx2, NEG, v2)
    m3 = jnp.max(v3, axis=1, keepdims=True)

    t = jnp.exp(m2 - m1)
    g1 = 1.0 / (1.0 + t)
    g2 = 1.0 - g1

    idx_ref[...] = jnp.concatenate([idx1, idx2], axis=1)
    gates_ref[...] = jnp.concatenate([g1, g2], axis=1)

    thr = jnp.where(noisy > m3, m3, m2)
    prob = _ncdf((clean - thr) / stddev)

    imp = (jnp.where(eio == idx1, g1, 0.0)
           + jnp.where(eio == idx2, g2, 0.0))
    acc_ref[0:1, :] += jnp.sum(prob, axis=0, keepdims=True)
    acc_ref[1:2, :] += jnp.sum(imp, axis=0, keepdims=True)

    @pl.when(i == pl.num_programs(0) - 1)
    def _():
        def cv_sq(x):
            mean = jnp.sum(x) / NUM_EXPERT
            var = jnp.sum((x - mean) ** 2) / (NUM_EXPERT - 1)
            return var / (mean * mean + 1e-10)

        total = cv_sq(acc_ref[1:2, :]) + cv_sq(acc_ref[0:1, :])
        loss_ref[...] = jnp.broadcast_to(total, (1, 1))


@jax.jit
def kernel(inp, w_gate, w_noise, noise):
    w_cat = jnp.concatenate([w_gate, w_noise], axis=1)
    grid = N_TOKENS // BLK
    idx, gates, loss = pl.pallas_call(
        _router_kernel,
        grid=(grid,),
        in_specs=[
            pl.BlockSpec((BLK, D_MODEL), lambda i: (i, 0)),
            pl.BlockSpec((D_MODEL, 2 * NUM_EXPERT), lambda i: (0, 0)),
            pl.BlockSpec((BLK, NUM_EXPERT), lambda i: (i, 0)),
        ],
        out_specs=[
            pl.BlockSpec((BLK, TOP_K), lambda i: (i, 0)),
            pl.BlockSpec((BLK, TOP_K), lambda i: (i, 0)),
            pl.BlockSpec((1, 1), lambda i: (0, 0)),
        ],
        out_shape=[
            jax.ShapeDtypeStruct((N_TOKENS, TOP_K), jnp.int32),
            jax.ShapeDtypeStruct((N_TOKENS, TOP_K), jnp.float32),
            jax.ShapeDtypeStruct((1, 1), jnp.float32),
        ],
        scratch_shapes=[pltpu.VMEM((2, NUM_EXPERT), jnp.float32)],
        compiler_params=pltpu.CompilerParams(
            dimension_semantics=("arbitrary",)),
    )(inp, w_cat, noise)
    return (idx.reshape(-1), gates.reshape(N_TOKENS, 1, TOP_K), loss[0, 0])


# speedup vs baseline: 6.0910x; 2.1754x over previous
import functools

import jax
import jax.numpy as jnp
from jax import lax
from jax.experimental import pallas as pl
from jax.experimental.pallas import tpu as pltpu

D_MODEL = 1024
NUM_EXPERT = 8
TOP_K = 2
N_TOKENS = 32768
NOISE_EPS = 0.01

BLK = 2048
NEG = -jnp.inf


def _ncdf(z):
    x = z * 0.7071067811865476
    a = jnp.abs(x)
    t = 1.0 / (1.0 + 0.3275911 * a)
    poly = t * (0.254829592 + t * (-0.284496736 + t * (1.421413741
                + t * (-1.453152027 + t * 1.061405429))))
    erf_a = 1.0 - poly * jnp.exp(-a * a)
    return 0.5 * (1.0 + jnp.where(x < 0, -erf_a, erf_a))


def _router_kernel(inp_ref, w_ref, noise_ref, idx_ref, gates_ref, loss_ref,
                   acc_ref):
    i = pl.program_id(0)

    @pl.when(i == 0)
    def _():
        acc_ref[...] = jnp.zeros_like(acc_ref)

    logits16 = lax.dot_general(
        w_ref[...], inp_ref[...],
        dimension_numbers=(((0,), (1,)), ((), ())),
        preferred_element_type=jnp.float32)
    clean = logits16[:NUM_EXPERT, :]
    raw = logits16[NUM_EXPERT:, :]
    stddev = (jnp.maximum(raw, 0.0)
              + jnp.log(1.0 + jnp.exp(-jnp.abs(raw))) + NOISE_EPS)
    noisy = clean + noise_ref[...] * stddev

    eio = lax.broadcasted_iota(jnp.int32, (NUM_EXPERT, BLK), 0)
    m1 = jnp.max(noisy, axis=0, keepdims=True)
    idx1 = jnp.min(jnp.where(noisy == m1, eio, NUM_EXPERT), axis=0,
                   keepdims=True)
    v2 = jnp.where(eio == idx1, NEG, noisy)
    m2 = jnp.max(v2, axis=0, keepdims=True)
    idx2 = jnp.min(jnp.where(v2 == m2, eio, NUM_EXPERT), axis=0,
                   keepdims=True)
    v3 = jnp.where(eio == idx2, NEG, v2)
    m3 = jnp.max(v3, axis=0, keepdims=True)

    t = jnp.exp(m2 - m1)
    g1 = 1.0 / (1.0 + t)
    g2 = 1.0 - g1

    idx_ref[...] = jnp.concatenate([idx1, idx2], axis=0)
    gates_ref[...] = jnp.concatenate([g1, g2], axis=0)

    thr = jnp.where(noisy > m3, m3, m2)
    prob = _ncdf((clean - thr) / stddev)

    imp = (jnp.where(eio == idx1, g1, 0.0)
           + jnp.where(eio == idx2, g2, 0.0))
    acc_ref[0] += jnp.sum(prob.reshape(NUM_EXPERT, BLK // 128, 128), axis=1)
    acc_ref[1] += jnp.sum(imp.reshape(NUM_EXPERT, BLK // 128, 128), axis=1)

    @pl.when(i == pl.num_programs(0) - 1)
    def _():
        def cv_sq(x):
            mean = jnp.sum(x) / NUM_EXPERT
            var = jnp.sum((x - mean) ** 2) / (NUM_EXPERT - 1)
            return var / (mean * mean + 1e-10)

        load = jnp.sum(acc_ref[0], axis=1)
        imp_t = jnp.sum(acc_ref[1], axis=1)
        loss_ref[...] = jnp.broadcast_to(cv_sq(imp_t) + cv_sq(load), (1, 1))


@jax.jit
def kernel(inp, w_gate, w_noise, noise):
    w_cat = jnp.concatenate([w_gate, w_noise], axis=1)
    grid = N_TOKENS // BLK
    idx, gates, loss = pl.pallas_call(
        _router_kernel,
        grid=(grid,),
        in_specs=[
            pl.BlockSpec((BLK, D_MODEL), lambda i: (i, 0)),
            pl.BlockSpec((D_MODEL, 2 * NUM_EXPERT), lambda i: (0, 0)),
            pl.BlockSpec((NUM_EXPERT, BLK), lambda i: (0, i)),
        ],
        out_specs=[
            pl.BlockSpec((TOP_K, BLK), lambda i: (0, i)),
            pl.BlockSpec((TOP_K, BLK), lambda i: (0, i)),
            pl.BlockSpec((1, 1), lambda i: (0, 0)),
        ],
        out_shape=[
            jax.ShapeDtypeStruct((TOP_K, N_TOKENS), jnp.int32),
            jax.ShapeDtypeStruct((TOP_K, N_TOKENS), jnp.float32),
            jax.ShapeDtypeStruct((1, 1), jnp.float32),
        ],
        scratch_shapes=[pltpu.VMEM((2, NUM_EXPERT, 128), jnp.float32)],
        compiler_params=pltpu.CompilerParams(
            dimension_semantics=("arbitrary",)),
    )(inp, w_cat, noise.T)
    return (idx.T.reshape(-1), gates.T.reshape(N_TOKENS, 1, TOP_K),
            loss[0, 0])
